# baseline (speedup 1.0000x reference)
import functools

import jax
import jax.numpy as jnp
from jax import lax
from jax.experimental import pallas as pl
from jax.experimental.pallas import tpu as pltpu
from jax.experimental.pallas import tpu_sc as plsc

_K_BY_HEAD = (10, 20, 40, 500)
_ROWS_PER_BLOCK = 256
_PROBES_PER_CHECK = 4
_MAX_CHECKS = 18
_SC_SLABS = 2


def _monotone_i32(b):
    return jnp.where(b >= 0, b, b ^ jnp.int32(0x7FFFFFFF))


def _unmap_f32(m):
    b = jnp.where(m >= 0, m, m ^ jnp.int32(0x7FFFFFFF))
    return jax.lax.bitcast_convert_type(b, jnp.float32)


def _topk_softmax_block(k_ref, x_ref, o_ref, lo_ref, hi_ref, clo_ref, chi_ref,
                        xt_ref):
    x = x_ref[0]
    k = k_ref[pl.program_id(0)]
    R, N = x.shape
    xt_ref[...] = x.T
    xt = xt_ref[...]

    xmin = jnp.min(xt, axis=0, keepdims=True)
    xmax = jnp.max(xt, axis=0, keepdims=True)
    lo_ref[...] = _monotone_i32(jax.lax.bitcast_convert_type(xmin, jnp.int32))
    hi_ref[...] = _monotone_i32(jax.lax.bitcast_convert_type(xmax, jnp.int32))
    clo_ref[...] = jnp.full((1, R), N, jnp.int32)
    chi_ref[...] = jnp.zeros((1, R), jnp.int32)

    def probe(u, state):
        lo, hi, clo, chi = state
        frozen = (chi == k) | (lo >= hi)
        lo_f = lo.astype(jnp.float32)
        hi_f = hi.astype(jnp.float32)
        frac = (clo - k).astype(jnp.float32) / jnp.maximum(
            (clo - chi).astype(jnp.float32), 1.0
        )
        mid_i = jnp.clip(
            (lo_f + (hi_f - lo_f) * frac).astype(jnp.int32), lo + 1, hi
        )
        mid_b = (lo >> 1) + (hi >> 1) + (lo & hi & 1) + ((lo ^ hi) & 1)
        mid = jnp.where(u % 2 == 0, mid_i, mid_b)
        mid = jnp.where(frozen, lo, mid)

        cnt = jnp.sum((xt >= _unmap_f32(mid)).astype(jnp.int32), axis=0,
                      keepdims=True)
        gt = cnt > k
        lo = jnp.where(frozen | ~gt, lo, mid)
        clo = jnp.where(frozen | ~gt, clo, cnt)
        hi = jnp.where(frozen | gt, hi, mid - 1)
        chi = jnp.where(frozen | gt, chi, cnt)
        return lo, hi, clo, chi

    def cond(c):
        return c < _MAX_CHECKS

    def body(c):
        state = (lo_ref[...], hi_ref[...], clo_ref[...], chi_ref[...])
        state = jax.lax.fori_loop(
            0, _PROBES_PER_CHECK,
            lambda u, s: probe(c * _PROBES_PER_CHECK + u, s),
            state, unroll=True,
        )
        lo, hi, clo, chi = state
        lo_ref[...] = lo
        hi_ref[...] = hi
        clo_ref[...] = clo
        chi_ref[...] = chi
        ndone = jnp.sum(((chi == k) | (lo >= hi)).astype(jnp.int32))
        return jnp.where(ndone < R, c + 1, _MAX_CHECKS + 1)

    jax.lax.while_loop(cond, body, jnp.int32(0))

    lo = lo_ref[...]
    hi = hi_ref[...]
    clo = clo_ref[...]
    chi = chi_ref[...]
    band_lo = _unmap_f32(lo)
    band_hi = _unmap_f32(hi + 1)
    definite = xt >= band_hi
    band = (xt >= band_lo) & jnp.logical_not(definite)
    j = k - chi
    bc = clo - chi

    keep = definite | (band & (bc == j))
    e = jnp.where(keep, jnp.exp(xt - xmax), 0.0)
    s = jnp.sum(e, axis=0, keepdims=True)
    o_ref[0] = (e / s).T

    @pl.when(jnp.any(bc > j))
    def _tie_fixup():
        C = 128
        nc = N // C
        bandr = (x >= band_lo.T) & (x < band_hi.T)
        b3 = bandr.astype(jnp.bfloat16).reshape(R, nc, C)
        i0 = jax.lax.broadcasted_iota(jnp.int32, (C, C), 0)
        i1 = jax.lax.broadcasted_iota(jnp.int32, (C, C), 1)
        tri = (i0 <= i1).astype(jnp.bfloat16)
        pc = jax.lax.dot_general(
            b3, tri, (((2,), (0,)), ((), ())),
            preferred_element_type=jnp.float32,
        )
        tot = pc[:, :, C - 1]
        s0 = jax.lax.broadcasted_iota(jnp.int32, (nc, nc), 0)
        s1 = jax.lax.broadcasted_iota(jnp.int32, (nc, nc), 1)
        stri = (s0 < s1).astype(jnp.float32)
        off = jax.lax.dot_general(
            tot, stri, (((1,), (0,)), ((), ())),
            preferred_element_type=jnp.float32,
        )
        prefix = (pc + off[:, :, None]).reshape(R, N)
        keep2 = (x >= band_hi.T) | (bandr & (prefix <= (k - chi.T).astype(jnp.float32)))
        e2 = jnp.where(keep2, jnp.exp(x - xmax.T), 0.0)
        s2 = jnp.sum(e2, axis=-1, keepdims=True)
        o_ref[0] = e2 / s2


def _make_sc_kernel(Q, N, first_slab, rows_per_slab):
    info = plsc.get_sparse_core_info()
    NW = info.num_cores * info.num_subcores
    gpw = Q // 16 // NW
    nch = N // 16

    mesh = plsc.VectorSubcoreMesh(core_axis_name="c", subcore_axis_name="s")

    @functools.partial(
        pl.kernel,
        mesh=mesh,
        compiler_params=pltpu.CompilerParams(use_tc_tiling_on_sc=False),
        out_type=jax.ShapeDtypeStruct((N, Q), jnp.float32),
        scratch_types=[
            pltpu.VMEM((N, 16), jnp.float32),
            pltpu.VMEM((N, 16), jnp.float32),
        ],
    )
    def sc_fn(xT_hbm, oT_hbm, in_v, out_v):
        wid = lax.axis_index("s") * info.num_cores + lax.axis_index("c")

        def grp_body(g, carry):
            r0 = (wid * gpw + g) * 16
            slab = first_slab + r0 // rows_per_slab
            h = slab % 4
            k = jnp.where(
                h == 0, 10, jnp.where(h == 1, 20, jnp.where(h == 2, 40, 500))
            ).astype(jnp.int32)
            pltpu.sync_copy(xT_hbm.at[:, pl.ds(r0, 16)], in_v)

            def probe(i, st):
                lo, hi = st
                mid = (lo >> 1) + (hi >> 1) + (lo & hi & 1) + ((lo ^ hi) & 1)
                t_f = _unmap_f32(mid)

                def cnt_chunk(c, acc):
                    for u in range(16):
                        v = in_v[c * 16 + u]
                        acc = acc + jnp.where(v >= t_f, 1, 0)
                    return acc

                cnt = lax.fori_loop(
                    0, nch, cnt_chunk, jnp.zeros((16,), jnp.int32)
                )
                ge = cnt >= k
                lo = jnp.where(ge, mid, lo)
                hi = jnp.where(ge, hi, mid - 1)
                return lo, hi

            lo0 = jnp.full((16,), -2139095041, jnp.int32)
            hi0 = jnp.full((16,), 2139095040, jnp.int32)
            lo, hi = lax.fori_loop(0, 32, probe, (lo0, hi0))
            t_f = _unmap_f32(lo)

            def e_chunk(c, acc):
                for u in range(16):
                    v = in_v[c * 16 + u]
                    e = jnp.where(v >= t_f, jnp.exp(v), 0.0)
                    out_v[c * 16 + u] = e
                    acc = acc + e
                return acc

            acc = lax.fori_loop(0, nch, e_chunk, jnp.zeros((16,), jnp.float32))
            rs = 1.0 / acc

            def s_chunk(c, cc):
                for u in range(16):
                    out_v[c * 16 + u] = out_v[c * 16 + u] * rs
                return cc

            lax.fori_loop(0, nch, s_chunk, jnp.int32(0))
            pltpu.sync_copy(out_v, oT_hbm.at[:, pl.ds(r0, 16)])
            return carry

        lax.fori_loop(0, gpw, grp_body, jnp.int32(0))

    return sc_fn


@jax.jit
def kernel(attention):
    B, H, M, N = attention.shape
    S = B * H
    x = attention.reshape(S, M, N)
    G = _SC_SLABS if (M == 2048 and N == 2048 and S == 16) else 0
    S_tc = S - G
    ks = jnp.tile(
        jnp.array([min(k, N) for k in _K_BY_HEAD], dtype=jnp.int32), B
    )[:S_tc]
    R = min(_ROWS_PER_BLOCK, M)
    nb = M // R

    grid_spec = pltpu.PrefetchScalarGridSpec(
        num_scalar_prefetch=1,
        grid=(S_tc, nb),
        in_specs=[
            pl.BlockSpec((1, R, N), lambda s, j, k_ref: (s, j, 0)),
        ],
        out_specs=pl.BlockSpec((1, R, N), lambda s, j, k_ref: (s, j, 0)),
        scratch_shapes=[
            pltpu.VMEM((1, R), jnp.int32),
            pltpu.VMEM((1, R), jnp.int32),
            pltpu.VMEM((1, R), jnp.int32),
            pltpu.VMEM((1, R), jnp.int32),
            pltpu.VMEM((N, R), jnp.float32),
        ],
    )
    if G > 0:
        sc_fn = _make_sc_kernel(G * M, N, S_tc, M)
        xT = x[S_tc:].reshape(G * M, N).T
        outT_sc = sc_fn(xT)
    out_tc = pl.pallas_call(
        _topk_softmax_block,
        grid_spec=grid_spec,
        out_shape=jax.ShapeDtypeStruct((S_tc, M, N), jnp.float32),
        compiler_params=pltpu.CompilerParams(
            dimension_semantics=("parallel", "parallel"),
        ),
    )(ks, x[:S_tc])
    if G == 0:
        return out_tc.reshape(B, H, M, N)
    out_sc = outT_sc.T.reshape(G, M, N)
    out = jnp.concatenate([out_tc, out_sc], axis=0)
    return out.reshape(B, H, M, N)
